# baseline (speedup 1.0000x reference)
import functools

import jax
import jax.numpy as jnp
from jax import lax
from jax.experimental import pallas as pl
from jax.experimental.pallas import tpu as pltpu
from jax.experimental.pallas import tpu_sc as plsc

_VOCAB = 50265
_HIDDEN = 768
_MAX_POS = 514
_PAD = 1
_EPS = 1e-05

_B = 64
_S = 512
_NTOK = _B * _S

_L = 16
_NC = 2
_NS = 16
_NW = _NC * _NS
_TPW = _NTOK // _NW
_ROWS_PW = _TPW // _S
_T = 16
_NCHK = _TPW // _T
_NRING = 4
_NCH = _HIDDEN // _L


def _comb_body(pos_ref, type_ref, o_ref):
    o_ref[...] = pos_ref[...] + type_ref[...]


def _make_comb(pos_emb, type_emb):
    return pl.pallas_call(
        _comb_body,
        out_shape=jax.ShapeDtypeStruct((_MAX_POS, _HIDDEN), jnp.float32),
    )(pos_emb, type_emb)


def _splat_f(x):
    return jnp.full((_L,), x, jnp.float32)


def _splat_i(x):
    return jnp.full((_L,), x, jnp.int32)


def _emb_body(ids_hbm, wtab_hbm, comb_hbm, gamma_hbm, beta_hbm, out_hbm,
              ids_v, pos_v, w0, w1, w2, w3, p0, p1, p2, p3,
              gam_v, bet_v, rstd_s, ms_s,
              sw0, sw1, sw2, sw3, sp0, sp1, sp2, sp3,
              so0, so1, so2, so3):
    wbufs = (w0, w1, w2, w3)
    pbufs = (p0, p1, p2, p3)
    sws = (sw0, sw1, sw2, sw3)
    sps = (sp0, sp1, sp2, sp3)
    sos = (so0, so1, so2, so3)
    sid = lax.axis_index("s")
    wid = sid * _NC + lax.axis_index("c")
    base = wid * _TPW

    pltpu.sync_copy(ids_hbm.at[pl.ds(base, _TPW)], ids_v)
    pltpu.sync_copy(gamma_hbm, gam_v)
    pltpu.sync_copy(beta_hbm, bet_v)

    def row_body(r, _):
        def cs_body(i, carry):
            off = r * _S + i * _L
            v = ids_v[pl.ds(off, _L)]
            nonpad = v != _splat_i(_PAD)
            m = nonpad.astype(jnp.int32)
            cs = plsc.cumsum(m) + _splat_i(carry)
            pos_v[pl.ds(off, _L)] = jnp.where(nonpad, cs + _splat_i(_PAD),
                                              _splat_i(_PAD))
            return carry + jnp.sum(m)
        lax.fori_loop(0, _S // _L, cs_body, jnp.int32(0))
        return 0

    lax.fori_loop(0, _ROWS_PW, row_body, 0)

    def g_issue(k, wb, pb, sw, sp):
        t0 = k * _T
        pltpu.async_copy(wtab_hbm.at[ids_v.at[pl.ds(t0, _T)]], wb, sw)
        pltpu.async_copy(comb_hbm.at[pos_v.at[pl.ds(t0, _T)]], pb, sp)

    def g_wait(wb, pb, sw, sp):
        pltpu.make_async_copy(
            wtab_hbm.at[ids_v.at[pl.ds(0, _T)]], wb, sw).wait()
        pltpu.make_async_copy(
            comb_hbm.at[pos_v.at[pl.ds(0, _T)]], pb, sp).wait()

    def s_issue(k, wb, so):
        pltpu.async_copy(wb, out_hbm.at[pl.ds(base + k * _T, _T)], so)

    def s_wait(wb, so):
        pltpu.make_async_copy(wb, out_hbm.at[pl.ds(base, _T)], so).wait()

    def compute_chunk(k, wb, pb):
        def tok_body(t, _):
            acc = jnp.zeros((_L,), jnp.float32)
            accsq = jnp.zeros((_L,), jnp.float32)
            for c in range(_NCH):
                sl = pl.ds(c * _L, _L)
                x = wb[t, sl] + pb[t, sl]
                wb[t, sl] = x
                acc = acc + x
                accsq = accsq + x * x
            mean = jnp.sum(acc) * (1.0 / _HIDDEN)
            var = jnp.sum(accsq) * (1.0 / _HIDDEN) - mean * mean
            vv = _splat_f(var + _EPS)
            bits = plsc.bitcast(vv, jnp.int32)
            y = plsc.bitcast(_splat_i(0x5F3759DF) - (bits >> _splat_i(1)),
                             jnp.float32)
            half_ve = _splat_f(0.5) * vv
            c15 = _splat_f(1.5)
            for _i in range(2):
                y = y * (c15 - half_ve * y * y)
            rstd = jnp.max(y)
            rstd_s[t] = rstd
            ms_s[t] = mean * rstd
            return 0

        lax.fori_loop(0, _T, tok_body, 0, unroll=2)

        cb = 8
        for blk in range(_NCH // cb):
            gs = [gam_v[pl.ds((blk * cb + i) * _L, _L)] for i in range(cb)]
            bs = [bet_v[pl.ds((blk * cb + i) * _L, _L)] for i in range(cb)]

            def t_body(t, _, blk=blk, gs=gs, bs=bs):
                rs = _splat_f(rstd_s[t])
                ms = _splat_f(ms_s[t])
                for i in range(cb):
                    sl = pl.ds((blk * cb + i) * _L, _L)
                    x = wb[t, sl]
                    wb[t, sl] = (x * rs - ms) * gs[i] + bs[i]
                return 0

            lax.fori_loop(0, _T, t_body, 0, unroll=4)

    g_issue(0, wbufs[0], pbufs[0], sws[0], sps[0])
    g_issue(1, wbufs[1], pbufs[1], sws[1], sps[1])

    def pipe_body(j, _):
        for i in range(_NRING):
            k = _NRING * j + i
            ip = (i + 2) % _NRING

            if i < 2:
                @pl.when(j >= 1)
                def _(ip=ip):
                    s_wait(wbufs[ip], sos[ip])
                g_issue(k + 2, wbufs[ip], pbufs[ip], sws[ip], sps[ip])
            else:
                s_wait(wbufs[ip], sos[ip])

                @pl.when(j < _NCHK // _NRING - 1)
                def _(k=k, ip=ip):
                    g_issue(k + 2, wbufs[ip], pbufs[ip], sws[ip], sps[ip])

            g_wait(wbufs[i], pbufs[i], sws[i], sps[i])
            compute_chunk(k, wbufs[i], pbufs[i])
            s_issue(k, wbufs[i], sos[i])
        return 0

    lax.fori_loop(0, _NCHK // _NRING, pipe_body, 0)
    s_wait(wbufs[2], sos[2])
    s_wait(wbufs[3], sos[3])


@jax.jit
def _run(ids, word_emb, comb, ln_gamma, ln_beta):
    mesh = plsc.VectorSubcoreMesh(core_axis_name="c", subcore_axis_name="s")
    f = functools.partial(
        pl.kernel,
        mesh=mesh,
        out_type=jax.ShapeDtypeStruct((_NTOK, _HIDDEN), jnp.float32),
        scratch_types=(
            [
                pltpu.VMEM((_TPW,), jnp.int32),
                pltpu.VMEM((_TPW,), jnp.int32),
            ]
            + [pltpu.VMEM((_T, _HIDDEN), jnp.float32)] * _NRING
            + [pltpu.VMEM((_T, _HIDDEN), jnp.float32)] * _NRING
            + [
                pltpu.VMEM((_HIDDEN,), jnp.float32),
                pltpu.VMEM((_HIDDEN,), jnp.float32),
                pltpu.SMEM((_T,), jnp.float32),
                pltpu.SMEM((_T,), jnp.float32),
            ]
            + [pltpu.SemaphoreType.DMA] * (3 * _NRING)
        ),
        compiler_params=pltpu.CompilerParams(needs_layout_passes=False),
    )(_emb_body)
    return f(ids, word_emb, comb, ln_gamma, ln_beta)


def kernel(input_ids, word_emb, pos_emb, type_emb, ln_gamma, ln_beta):
    comb = _make_comb(pos_emb, type_emb)
    ids = input_ids.reshape(_NTOK).astype(jnp.int32)
    out = _run(ids, word_emb, comb, ln_gamma, ln_beta)
    return out.reshape(_B, _S, _HIDDEN)


# speedup vs baseline: 1.2585x; 1.2585x over previous
import functools

import jax
import jax.numpy as jnp
from jax import lax
from jax.experimental import pallas as pl
from jax.experimental.pallas import tpu as pltpu
from jax.experimental.pallas import tpu_sc as plsc

_VOCAB = 50265
_HIDDEN = 768
_MAX_POS = 514
_PAD = 1
_EPS = 1e-05

_B = 64
_S = 512
_NTOK = _B * _S

_L = 16
_NC = 2
_NS = 16
_NW = _NC * _NS
_TPW = _NTOK // _NW
_ROWS_PW = _TPW // _S
_T = 16
_NCHK = _TPW // _T
_NRING = 4
_NCH = _HIDDEN // _L


def _comb_body(pos_ref, type_ref, o_ref):
    o_ref[...] = pos_ref[...] + type_ref[...]


def _make_comb(pos_emb, type_emb):
    return pl.pallas_call(
        _comb_body,
        out_shape=jax.ShapeDtypeStruct((_MAX_POS, _HIDDEN), jnp.float32),
    )(pos_emb, type_emb)


def _splat_f(x):
    return jnp.full((_L,), x, jnp.float32)


def _splat_i(x):
    return jnp.full((_L,), x, jnp.int32)


def _emb_body(ids_hbm, wtab_hbm, comb_hbm, gamma_hbm, beta_hbm, out_hbm,
              ids_v, pos_v, w0, w1, w2, w3, p0, p1, p2, p3,
              gam_v, bet_v, rstd_s, ms_s,
              sw0, sw1, sw2, sw3, sp0, sp1, sp2, sp3,
              so0, so1, so2, so3):
    wbufs = (w0, w1, w2, w3)
    pbufs = (p0, p1, p2, p3)
    sws = (sw0, sw1, sw2, sw3)
    sps = (sp0, sp1, sp2, sp3)
    sos = (so0, so1, so2, so3)
    sid = lax.axis_index("s")
    wid = sid * _NC + lax.axis_index("c")
    base = wid * _TPW

    pltpu.sync_copy(ids_hbm.at[pl.ds(base, _TPW)], ids_v)
    pltpu.sync_copy(gamma_hbm, gam_v)
    pltpu.sync_copy(beta_hbm, bet_v)

    def row_body(r, _):
        def cs_body(i, carry):
            off = r * _S + i * _L
            v = ids_v[pl.ds(off, _L)]
            nonpad = v != _splat_i(_PAD)
            m = nonpad.astype(jnp.int32)
            cs = plsc.cumsum(m) + _splat_i(carry)
            pos_v[pl.ds(off, _L)] = jnp.where(nonpad, cs + _splat_i(_PAD),
                                              _splat_i(_PAD))
            return carry + jnp.sum(m)
        lax.fori_loop(0, _S // _L, cs_body, jnp.int32(0))
        return 0

    lax.fori_loop(0, _ROWS_PW, row_body, 0)

    def g_issue(k, wb, pb, sw, sp):
        t0 = k * _T
        pltpu.async_copy(wtab_hbm.at[ids_v.at[pl.ds(t0, _T)]], wb, sw)
        pltpu.async_copy(comb_hbm.at[pos_v.at[pl.ds(t0, _T)]], pb, sp)

    def g_wait(wb, pb, sw, sp):
        pltpu.make_async_copy(
            wtab_hbm.at[ids_v.at[pl.ds(0, _T)]], wb, sw).wait()
        pltpu.make_async_copy(
            comb_hbm.at[pos_v.at[pl.ds(0, _T)]], pb, sp).wait()

    def s_issue(k, wb, so):
        pltpu.async_copy(wb, out_hbm.at[pl.ds(base + k * _T, _T)], so)

    def s_wait(wb, so):
        pltpu.make_async_copy(wb, out_hbm.at[pl.ds(base, _T)], so).wait()

    def compute_chunk(k, wb, pb):
        def tok_body(t, _):
            acc = jnp.zeros((_L,), jnp.float32)
            accsq = jnp.zeros((_L,), jnp.float32)
            for c in range(_NCH):
                sl = pl.ds(c * _L, _L)
                x = wb[t, sl] + pb[t, sl]
                wb[t, sl] = x
                acc = acc + x
                accsq = accsq + x * x
            mean = jnp.sum(acc) * (1.0 / _HIDDEN)
            var = jnp.sum(accsq) * (1.0 / _HIDDEN) - mean * mean
            vv = _splat_f(var + _EPS)
            bits = plsc.bitcast(vv, jnp.int32)
            y = plsc.bitcast(_splat_i(0x5F3759DF) - (bits >> _splat_i(1)),
                             jnp.float32)
            half_ve = _splat_f(0.5) * vv
            c15 = _splat_f(1.5)
            for _i in range(2):
                y = y * (c15 - half_ve * y * y)
            rstd = jnp.max(y)
            rstd_s[t] = rstd
            ms_s[t] = mean * rstd
            return 0

        lax.fori_loop(0, _T, tok_body, 0, unroll=2)

        cb = 8
        for blk in range(_NCH // cb):
            gs = [gam_v[pl.ds((blk * cb + i) * _L, _L)] for i in range(cb)]
            bs = [bet_v[pl.ds((blk * cb + i) * _L, _L)] for i in range(cb)]

            def t_body(t, _, blk=blk, gs=gs, bs=bs):
                rs = _splat_f(rstd_s[t])
                ms = _splat_f(ms_s[t])
                for i in range(cb):
                    sl = pl.ds((blk * cb + i) * _L, _L)
                    x = wb[t, sl]
                    wb[t, sl] = (x * rs - ms) * gs[i] + bs[i]
                return 0

            lax.fori_loop(0, _T, t_body, 0, unroll=2)

    g_issue(0, wbufs[0], pbufs[0], sws[0], sps[0])
    g_issue(1, wbufs[1], pbufs[1], sws[1], sps[1])

    def pipe_body(j, _):
        for i in range(_NRING):
            k = _NRING * j + i
            ip = (i + 2) % _NRING

            if i < 2:
                @pl.when(j >= 1)
                def _(ip=ip):
                    s_wait(wbufs[ip], sos[ip])
                g_issue(k + 2, wbufs[ip], pbufs[ip], sws[ip], sps[ip])
            else:
                s_wait(wbufs[ip], sos[ip])

                @pl.when(j < _NCHK // _NRING - 1)
                def _(k=k, ip=ip):
                    g_issue(k + 2, wbufs[ip], pbufs[ip], sws[ip], sps[ip])

            g_wait(wbufs[i], pbufs[i], sws[i], sps[i])
            compute_chunk(k, wbufs[i], pbufs[i])
            s_issue(k, wbufs[i], sos[i])
        return 0

    lax.fori_loop(0, _NCHK // _NRING, pipe_body, 0)
    s_wait(wbufs[2], sos[2])
    s_wait(wbufs[3], sos[3])


@jax.jit
def _run(ids, word_emb, comb, ln_gamma, ln_beta):
    mesh = plsc.VectorSubcoreMesh(core_axis_name="c", subcore_axis_name="s")
    f = functools.partial(
        pl.kernel,
        mesh=mesh,
        out_type=jax.ShapeDtypeStruct((_NTOK, _HIDDEN), jnp.float32),
        scratch_types=(
            [
                pltpu.VMEM((_TPW,), jnp.int32),
                pltpu.VMEM((_TPW,), jnp.int32),
            ]
            + [pltpu.VMEM((_T, _HIDDEN), jnp.float32)] * _NRING
            + [pltpu.VMEM((_T, _HIDDEN), jnp.float32)] * _NRING
            + [
                pltpu.VMEM((_HIDDEN,), jnp.float32),
                pltpu.VMEM((_HIDDEN,), jnp.float32),
                pltpu.SMEM((_T,), jnp.float32),
                pltpu.SMEM((_T,), jnp.float32),
            ]
            + [pltpu.SemaphoreType.DMA] * (3 * _NRING)
        ),
        compiler_params=pltpu.CompilerParams(needs_layout_passes=False),
    )(_emb_body)
    return f(ids, word_emb, comb, ln_gamma, ln_beta)


def kernel(input_ids, word_emb, pos_emb, type_emb, ln_gamma, ln_beta):
    comb = _make_comb(pos_emb, type_emb)
    ids = input_ids.reshape(_NTOK).astype(jnp.int32)
    out = _run(ids, word_emb, comb, ln_gamma, ln_beta)
    return out.reshape(_B, _S, _HIDDEN)
